# baseline (speedup 1.0000x reference)
import functools

import jax
import jax.numpy as jnp
from jax.experimental import pallas as pl
from jax.experimental.pallas import tpu as pltpu

B = 4
N = 10000
E = 320000
DV = 128
DE = 16
DC = 32

EW = 128
ER = E * DE // EW

VCH = 1000
ECH = 1000
NV = N // VCH
NE = ER // ECH
NSTEPS = NV + NE


def _tc_kernel(ctx_ref, v_ref, e_ref, w_ref, b_ref, out_ref, acc_v, acc_e):
    i = pl.program_id(0)

    @pl.when(i == 0)
    def _init():
        acc_v[...] = jnp.zeros_like(acc_v)
        acc_e[...] = jnp.zeros_like(acc_e)

    @pl.when(i < NV)
    def _vstep():
        acc_v[...] += jnp.sum(v_ref[...], axis=1)

    @pl.when(i >= NV)
    def _estep():
        acc_e[...] += jnp.sum(e_ref[...], axis=1)

    @pl.when(i == NSTEPS - 1)
    def _final():
        v_agg = acc_v[...] * (1.0 / N)
        e128 = acc_e[...]
        e_agg = jnp.zeros((B, DE), jnp.float32)
        for k in range(EW // DE):
            e_agg = e_agg + e128[:, k * DE:(k + 1) * DE]
        e_agg = e_agg * (1.0 / E)
        ctx = ctx_ref[...][:, 0, :]
        w = w_ref[...]
        out = (
            jnp.dot(ctx, w[:DC], preferred_element_type=jnp.float32)
            + jnp.dot(v_agg, w[DC:DC + DV], preferred_element_type=jnp.float32)
            + jnp.dot(e_agg, w[DC + DV:], preferred_element_type=jnp.float32)
            + b_ref[...][None, :]
        )
        out_ref[...] = out[:, None, :]


@jax.jit
def kernel(context, vertex_data, edge_data, W, b):
    edge_r = edge_data.reshape(B, ER, EW)
    grid = (NSTEPS,)
    return pl.pallas_call(
        _tc_kernel,
        grid=grid,
        in_specs=[
            pl.BlockSpec((B, 1, DC), lambda i: (0, 0, 0)),
            pl.BlockSpec((B, VCH, DV), lambda i: (0, jnp.minimum(i, NV - 1), 0)),
            pl.BlockSpec((B, ECH, EW), lambda i: (0, jnp.maximum(i - NV, 0), 0)),
            pl.BlockSpec((DC + DV + DE, DC), lambda i: (0, 0)),
            pl.BlockSpec((DC,), lambda i: (0,)),
        ],
        out_specs=pl.BlockSpec((B, 1, DC), lambda i: (0, 0, 0)),
        out_shape=jax.ShapeDtypeStruct((B, 1, DC), jnp.float32),
        scratch_shapes=[
            pltpu.VMEM((B, DV), jnp.float32),
            pltpu.VMEM((B, EW), jnp.float32),
        ],
        compiler_params=pltpu.CompilerParams(
            dimension_semantics=("arbitrary",),
        ),
    )(context, vertex_data, edge_r, W, b)
